# baseline (device time: 27519 ns/iter reference)
import jax
import jax.numpy as jnp
from jax import lax
from jax.experimental import pallas as pl
from jax.experimental.pallas import tpu as pltpu

N_DEV = 4
N_LAYERS = 3
N_CHUNKS = 2


def kernel(x, Win0, Wout0, Win1, Wout1, Win2, Wout2):
    b, d = x.shape
    bc = b // N_CHUNKS

    def body(x_ref, win0_ref, wout0_ref, win1_ref, wout1_ref, win2_ref,
             wout2_ref, out_ref, stage_ref, comm_ref, send_sems, recv_sems):
        my = lax.axis_index("i")

        barrier_sem = pltpu.get_barrier_semaphore()
        for dist in range(1, N_DEV):
            pl.semaphore_signal(
                barrier_sem, inc=1,
                device_id=((my + dist) % N_DEV,),
                device_id_type=pl.DeviceIdType.MESH,
            )
        pl.semaphore_wait(barrier_sem, N_DEV - 1)

        wins = [win0_ref, win1_ref, win2_ref]
        wouts = [wout0_ref, wout1_ref, wout2_ref]

        def fwd(x_c, l):
            h = jnp.maximum(
                jnp.dot(x_c, wins[l][:, :], preferred_element_type=jnp.float32),
                0.0,
            )
            return jnp.dot(h, wouts[l][:, :], preferred_element_type=jnp.float32)

        rdmas = {}

        def send_chunk(l, c, partial):
            stage_ref[l, c, :, :] = partial
            for dist in range(1, N_DEV):
                rdma = pltpu.make_async_remote_copy(
                    src_ref=stage_ref.at[l, c],
                    dst_ref=comm_ref.at[l, c, dist - 1],
                    send_sem=send_sems.at[l, c, dist - 1],
                    recv_sem=recv_sems.at[l, c, dist - 1],
                    device_id=((my + dist) % N_DEV,),
                    device_id_type=pl.DeviceIdType.MESH,
                )
                rdma.start()
                rdmas[(l, c, dist)] = rdma

        parts = []
        for c in range(N_CHUNKS):
            p = fwd(x_ref[pl.ds(c * bc, bc), :], 0)
            send_chunk(0, c, p)
            parts.append(p)

        for l in range(N_LAYERS):
            next_parts = []
            for c in range(N_CHUNKS):
                for dist in range(1, N_DEV):
                    rdmas[(l, c, dist)].wait_recv()
                red = (parts[c] + comm_ref[l, c, 0] + comm_ref[l, c, 1]
                       + comm_ref[l, c, 2])
                if l < N_LAYERS - 1:
                    p = fwd(red, l + 1)
                    send_chunk(l + 1, c, p)
                    next_parts.append(p)
                else:
                    out_ref[pl.ds(c * bc, bc), :] = red
            parts = next_parts

        for rdma in rdmas.values():
            rdma.wait_send()

    return pl.pallas_call(
        body,
        out_shape=jax.ShapeDtypeStruct((b, d), jnp.float32),
        in_specs=[pl.BlockSpec(memory_space=pltpu.VMEM)] * 7,
        out_specs=pl.BlockSpec(memory_space=pltpu.VMEM),
        scratch_shapes=[
            pltpu.VMEM((N_LAYERS, N_CHUNKS, bc, d), jnp.float32),
            pltpu.VMEM((N_LAYERS, N_CHUNKS, N_DEV - 1, bc, d), jnp.float32),
            pltpu.SemaphoreType.DMA((N_LAYERS, N_CHUNKS, N_DEV - 1)),
            pltpu.SemaphoreType.DMA((N_LAYERS, N_CHUNKS, N_DEV - 1)),
        ],
        compiler_params=pltpu.CompilerParams(collective_id=0),
    )(x, Win0, Wout0, Win1, Wout1, Win2, Wout2)


# device time: 24796 ns/iter; 1.1098x vs baseline; 1.1098x over previous
import jax
import jax.numpy as jnp
from jax import lax
from jax.experimental import pallas as pl
from jax.experimental.pallas import tpu as pltpu

N_DEV = 4
N_LAYERS = 3
N_CHUNKS = 2
WIRE_DT = jnp.bfloat16


def kernel(x, Win0, Wout0, Win1, Wout1, Win2, Wout2):
    b, d = x.shape
    bc = b // N_CHUNKS

    def body(x_ref, win0_ref, wout0_ref, win1_ref, wout1_ref, win2_ref,
             wout2_ref, out_ref, stage_ref, comm_ref, send_sems, recv_sems):
        my = lax.axis_index("i")

        barrier_sem = pltpu.get_barrier_semaphore()
        for dist in range(1, N_DEV):
            pl.semaphore_signal(
                barrier_sem, inc=1,
                device_id=((my + dist) % N_DEV,),
                device_id_type=pl.DeviceIdType.MESH,
            )
        pl.semaphore_wait(barrier_sem, N_DEV - 1)

        wins = [win0_ref, win1_ref, win2_ref]
        wouts = [wout0_ref, wout1_ref, wout2_ref]

        def fwd(x_c, l):
            h = jnp.maximum(
                jnp.dot(x_c, wins[l][:, :], preferred_element_type=jnp.float32),
                0.0,
            )
            return jnp.dot(h, wouts[l][:, :], preferred_element_type=jnp.float32)

        rdmas = {}

        def send_chunk(l, c, partial):
            stage_ref[l, c, :, :] = partial.astype(WIRE_DT)
            for dist in range(1, N_DEV):
                rdma = pltpu.make_async_remote_copy(
                    src_ref=stage_ref.at[l, c],
                    dst_ref=comm_ref.at[l, c, dist - 1],
                    send_sem=send_sems.at[l, c, dist - 1],
                    recv_sem=recv_sems.at[l, c, dist - 1],
                    device_id=((my + dist) % N_DEV,),
                    device_id_type=pl.DeviceIdType.MESH,
                )
                rdma.start()
                rdmas[(l, c, dist)] = rdma

        parts = []
        for c in range(N_CHUNKS):
            p = fwd(x_ref[pl.ds(c * bc, bc), :], 0)
            send_chunk(0, c, p)
            parts.append(p)

        for l in range(N_LAYERS):
            next_parts = []
            for c in range(N_CHUNKS):
                for dist in range(1, N_DEV):
                    rdmas[(l, c, dist)].wait_recv()
                red = parts[c]
                for k in range(N_DEV - 1):
                    red = red + comm_ref[l, c, k].astype(jnp.float32)
                if l < N_LAYERS - 1:
                    p = fwd(red, l + 1)
                    send_chunk(l + 1, c, p)
                    next_parts.append(p)
                else:
                    out_ref[pl.ds(c * bc, bc), :] = red
            parts = next_parts

        for rdma in rdmas.values():
            rdma.wait_send()

    return pl.pallas_call(
        body,
        out_shape=jax.ShapeDtypeStruct((b, d), jnp.float32),
        in_specs=[pl.BlockSpec(memory_space=pltpu.VMEM)] * 7,
        out_specs=pl.BlockSpec(memory_space=pltpu.VMEM),
        scratch_shapes=[
            pltpu.VMEM((N_LAYERS, N_CHUNKS, bc, d), WIRE_DT),
            pltpu.VMEM((N_LAYERS, N_CHUNKS, N_DEV - 1, bc, d), WIRE_DT),
            pltpu.SemaphoreType.DMA((N_LAYERS, N_CHUNKS, N_DEV - 1)),
            pltpu.SemaphoreType.DMA((N_LAYERS, N_CHUNKS, N_DEV - 1)),
        ],
        compiler_params=pltpu.CompilerParams(collective_id=0),
    )(x, Win0, Wout0, Win1, Wout1, Win2, Wout2)


# device time: 24331 ns/iter; 1.1310x vs baseline; 1.0191x over previous
import jax
import jax.numpy as jnp
from jax import lax
from jax.experimental import pallas as pl
from jax.experimental.pallas import tpu as pltpu

N_DEV = 4
N_LAYERS = 3
N_CHUNKS = 4
WIRE_DT = jnp.bfloat16


def kernel(x, Win0, Wout0, Win1, Wout1, Win2, Wout2):
    b, d = x.shape
    bc = b // N_CHUNKS

    def body(x_ref, win0_ref, wout0_ref, win1_ref, wout1_ref, win2_ref,
             wout2_ref, out_ref, stage_ref, comm_ref, send_sems, recv_sems):
        my = lax.axis_index("i")

        barrier_sem = pltpu.get_barrier_semaphore()
        for dist in range(1, N_DEV):
            pl.semaphore_signal(
                barrier_sem, inc=1,
                device_id=((my + dist) % N_DEV,),
                device_id_type=pl.DeviceIdType.MESH,
            )
        pl.semaphore_wait(barrier_sem, N_DEV - 1)

        wins = [win0_ref, win1_ref, win2_ref]
        wouts = [wout0_ref, wout1_ref, wout2_ref]

        def fwd(x_c, l):
            h = jnp.maximum(
                jnp.dot(x_c, wins[l][:, :], preferred_element_type=jnp.float32),
                0.0,
            )
            return jnp.dot(h, wouts[l][:, :], preferred_element_type=jnp.float32)

        rdmas = {}

        def send_chunk(l, c, partial):
            stage_ref[l, c, :, :] = partial.astype(WIRE_DT)
            for dist in (2, 1, 3):
                rdma = pltpu.make_async_remote_copy(
                    src_ref=stage_ref.at[l, c],
                    dst_ref=comm_ref.at[l, c, dist - 1],
                    send_sem=send_sems.at[l, c, dist - 1],
                    recv_sem=recv_sems.at[l, c, dist - 1],
                    device_id=((my + dist) % N_DEV,),
                    device_id_type=pl.DeviceIdType.MESH,
                )
                rdma.start()
                rdmas[(l, c, dist)] = rdma

        parts = []
        for c in range(N_CHUNKS):
            p = fwd(x_ref[pl.ds(c * bc, bc), :], 0)
            send_chunk(0, c, p)
            parts.append(p)

        for l in range(N_LAYERS):
            next_parts = []
            for c in range(N_CHUNKS):
                for dist in range(1, N_DEV):
                    rdmas[(l, c, dist)].wait_recv()
                red = parts[c]
                for k in range(N_DEV - 1):
                    red = red + comm_ref[l, c, k].astype(jnp.float32)
                if l < N_LAYERS - 1:
                    p = fwd(red, l + 1)
                    send_chunk(l + 1, c, p)
                    next_parts.append(p)
                else:
                    out_ref[pl.ds(c * bc, bc), :] = red
            parts = next_parts

        for rdma in rdmas.values():
            rdma.wait_send()

    return pl.pallas_call(
        body,
        out_shape=jax.ShapeDtypeStruct((b, d), jnp.float32),
        in_specs=[pl.BlockSpec(memory_space=pltpu.VMEM)] * 7,
        out_specs=pl.BlockSpec(memory_space=pltpu.VMEM),
        scratch_shapes=[
            pltpu.VMEM((N_LAYERS, N_CHUNKS, bc, d), WIRE_DT),
            pltpu.VMEM((N_LAYERS, N_CHUNKS, N_DEV - 1, bc, d), WIRE_DT),
            pltpu.SemaphoreType.DMA((N_LAYERS, N_CHUNKS, N_DEV - 1)),
            pltpu.SemaphoreType.DMA((N_LAYERS, N_CHUNKS, N_DEV - 1)),
        ],
        compiler_params=pltpu.CompilerParams(collective_id=0),
    )(x, Win0, Wout0, Win1, Wout1, Win2, Wout2)
